# baseline (speedup 1.0000x reference)
import functools

import jax
import jax.numpy as jnp
from jax import lax
from jax.experimental import pallas as pl
from jax.experimental.pallas import tpu as pltpu
from jax.experimental.pallas import tpu_sc as plsc

_B = 4096 * 200
_D = 64
_NC, _NS = 2, 16
_NW = _NC * _NS
_BPW = _B // _NW
_C = 256
_NCHUNK = _BPW // _C

_mesh = plsc.VectorSubcoreMesh(core_axis_name="c", subcore_axis_name="s")


@functools.partial(
    pl.kernel,
    out_type=jax.ShapeDtypeStruct((_B, _D), jnp.float32),
    mesh=_mesh,
    scratch_types=[
        pltpu.VMEM((_C,), jnp.int32),
        pltpu.VMEM((_C,), jnp.int32),
        pltpu.VMEM((_C,), jnp.int32),
        pltpu.VMEM((_C, 2 * _D), jnp.float32),
        pltpu.VMEM((_C, _D), jnp.float32),
        pltpu.SemaphoreType.DMA,
    ],
    compiler_params=pltpu.CompilerParams(needs_layout_passes=False),
)
def _embed(idx_hbm, table_hbm, out_hbm, idx_v, line_v, half_v, rows_v,
           store_v, sem):
    wid = lax.axis_index("s") * _NC + lax.axis_index("c")
    wbase = wid * _BPW
    iota16 = lax.iota(jnp.int32, 16)

    def body(g, carry):
        base = wbase + g * _C
        pltpu.sync_copy(idx_hbm.at[pl.ds(base, _C)], idx_v)

        def prep_body(j, c):
            v = idx_v[pl.ds(j * 16, 16)]
            line_v[pl.ds(j * 16, 16)] = lax.shift_right_logical(v, 1)
            half_v[pl.ds(j * 16, 16)] = (v & 1) * _D
            return c

        lax.fori_loop(0, _C // 16, prep_body, 0)

        pltpu.async_copy(table_hbm.at[line_v], rows_v, sem).wait()

        def extract_body(j, c):
            row16 = j * 16 + iota16
            par16 = half_v[pl.ds(j * 16, 16)]
            for col in range(_D):
                v = plsc.load_gather(rows_v, [row16, par16 + col])
                plsc.store_scatter(
                    store_v, [row16, jnp.full((16,), col, jnp.int32)], v
                )
            return c

        lax.fori_loop(0, _C // 16, extract_body, 0)

        pltpu.sync_copy(store_v, out_hbm.at[pl.ds(base, _C)])
        return carry

    lax.fori_loop(0, _NCHUNK, body, 0)


def kernel(inputs, table):
    idx = inputs.reshape(-1).astype(jnp.int32)
    table2 = table.reshape(table.shape[0] // 2, 2 * table.shape[1])
    out = _embed(idx, table2)
    return out.reshape(inputs.shape + (table.shape[1],))


# speedup vs baseline: 2.5207x; 2.5207x over previous
import functools

import jax
import jax.numpy as jnp
from jax import lax
from jax.experimental import pallas as pl
from jax.experimental.pallas import tpu as pltpu
from jax.experimental.pallas import tpu_sc as plsc

_B = 4096 * 200
_D = 64
_NC, _NS = 2, 16
_NW = _NC * _NS
_BPW = _B // _NW
_C = 512
_NCHUNK = _BPW // _C

_mesh = plsc.VectorSubcoreMesh(core_axis_name="c", subcore_axis_name="s")


@functools.partial(
    pl.kernel,
    out_type=jax.ShapeDtypeStruct((_B, 2 * _D), jnp.float32),
    mesh=_mesh,
    scratch_types=[
        pltpu.VMEM((_C,), jnp.int32),
        pltpu.VMEM((_C, 2 * _D), jnp.float32),
        pltpu.SemaphoreType.DMA,
    ],
    compiler_params=pltpu.CompilerParams(needs_layout_passes=False),
)
def _embed(idx_hbm, table_hbm, out_hbm, idx_v, rows_v, sem):
    wid = lax.axis_index("s") * _NC + lax.axis_index("c")
    wbase = wid * _BPW

    def body(g, carry):
        base = wbase + g * _C
        pltpu.sync_copy(idx_hbm.at[pl.ds(base, _C)], idx_v)
        pltpu.async_copy(table_hbm.at[idx_v], rows_v, sem).wait()
        pltpu.sync_copy(rows_v, out_hbm.at[pl.ds(base, _C)])
        return carry

    lax.fori_loop(0, _NCHUNK, body, 0)


def kernel(inputs, table):
    idx = inputs.reshape(-1).astype(jnp.int32)
    lines = jnp.concatenate([table, table], axis=1)
    out = _embed(idx, lines)
    return out[:, : table.shape[1]].reshape(inputs.shape + (table.shape[1],))


# speedup vs baseline: 2.5766x; 1.0222x over previous
import functools

import jax
import jax.numpy as jnp
from jax import lax
from jax.experimental import pallas as pl
from jax.experimental.pallas import tpu as pltpu
from jax.experimental.pallas import tpu_sc as plsc

_B = 4096 * 200
_D = 64
_NC, _NS = 2, 16
_NW = _NC * _NS
_BPW = _B // _NW
_C = 800
_NCHUNK = _BPW // _C

_mesh = plsc.VectorSubcoreMesh(core_axis_name="c", subcore_axis_name="s")


@functools.partial(
    pl.kernel,
    out_type=jax.ShapeDtypeStruct((_B, 2 * _D), jnp.float32),
    mesh=_mesh,
    scratch_types=[
        pltpu.VMEM((_C,), jnp.int32),
        pltpu.VMEM((_C, 2 * _D), jnp.float32),
        pltpu.SemaphoreType.DMA,
    ],
    compiler_params=pltpu.CompilerParams(needs_layout_passes=False),
)
def _embed(idx_hbm, table_hbm, out_hbm, idx_v, rows_v, sem):
    wid = lax.axis_index("s") * _NC + lax.axis_index("c")
    wbase = wid * _BPW

    def body(g, carry):
        base = wbase + g * _C
        pltpu.sync_copy(idx_hbm.at[pl.ds(base, _C)], idx_v)
        pltpu.async_copy(table_hbm.at[idx_v], rows_v, sem).wait()
        pltpu.sync_copy(rows_v, out_hbm.at[pl.ds(base, _C)])
        return carry

    lax.fori_loop(0, _NCHUNK, body, 0)


def kernel(inputs, table):
    idx = inputs.reshape(-1).astype(jnp.int32)
    lines = jnp.concatenate([table, table], axis=1)
    out = _embed(idx, lines)
    return out[:, : table.shape[1]].reshape(inputs.shape + (table.shape[1],))


# speedup vs baseline: 2.6143x; 1.0146x over previous
import functools

import jax
import jax.numpy as jnp
from jax import lax
from jax.experimental import pallas as pl
from jax.experimental.pallas import tpu as pltpu
from jax.experimental.pallas import tpu_sc as plsc

_B = 4096 * 200
_D = 64
_NC, _NS = 2, 16
_NW = _NC * _NS
_BPW = _B // _NW
_C = 400
_NCHUNK = _BPW // _C
_K = _NCHUNK // 2

_mesh = plsc.VectorSubcoreMesh(core_axis_name="c", subcore_axis_name="s")


@functools.partial(
    pl.kernel,
    out_type=jax.ShapeDtypeStruct((_B, 2 * _D), jnp.float32),
    mesh=_mesh,
    scratch_types=[
        pltpu.VMEM((_C,), jnp.int32),
        pltpu.VMEM((_C,), jnp.int32),
        pltpu.VMEM((_C, 2 * _D), jnp.float32),
        pltpu.VMEM((_C, 2 * _D), jnp.float32),
        pltpu.SemaphoreType.DMA,
        pltpu.SemaphoreType.DMA,
        pltpu.SemaphoreType.DMA,
        pltpu.SemaphoreType.DMA,
        pltpu.SemaphoreType.DMA,
        pltpu.SemaphoreType.DMA,
    ],
    compiler_params=pltpu.CompilerParams(needs_layout_passes=False),
)
def _embed(idx_hbm, table_hbm, out_hbm, idx0, idx1, rows0, rows1,
           isem0, isem1, gsem0, gsem1, ssem0, ssem1):
    wid = lax.axis_index("s") * _NC + lax.axis_index("c")
    wbase = wid * _BPW

    def idx_slice(g):
        return idx_hbm.at[pl.ds(wbase + g * _C, _C)]

    def out_slice(g):
        return out_hbm.at[pl.ds(wbase + g * _C, _C)]

    pltpu.async_copy(idx_slice(0), idx0, isem0)
    pltpu.async_copy(idx_slice(1), idx1, isem1)
    pltpu.make_async_copy(idx_slice(0), idx0, isem0).wait()
    pltpu.async_copy(table_hbm.at[idx0], rows0, gsem0)

    def body(k, carry):
        g0 = 2 * k

        pltpu.make_async_copy(table_hbm.at[idx0], rows0, gsem0).wait()
        pltpu.async_copy(rows0, out_slice(g0), ssem0)

        @pl.when(k < _K - 1)
        def _():
            pltpu.async_copy(idx_slice(g0 + 2), idx0, isem0)

        @pl.when(k > 0)
        def _():
            pltpu.make_async_copy(rows1, out_slice(g0), ssem1).wait()

        pltpu.make_async_copy(idx_slice(g0 + 1), idx1, isem1).wait()
        pltpu.async_copy(table_hbm.at[idx1], rows1, gsem1)
        pltpu.make_async_copy(table_hbm.at[idx1], rows1, gsem1).wait()
        pltpu.async_copy(rows1, out_slice(g0 + 1), ssem1)

        @pl.when(k < _K - 1)
        def _():
            pltpu.async_copy(idx_slice(g0 + 3), idx1, isem1)
            pltpu.make_async_copy(rows0, out_slice(g0), ssem0).wait()
            pltpu.make_async_copy(idx_slice(g0 + 2), idx0, isem0).wait()
            pltpu.async_copy(table_hbm.at[idx0], rows0, gsem0)

        return carry

    lax.fori_loop(0, _K, body, 0)

    pltpu.make_async_copy(rows0, out_slice(0), ssem0).wait()
    pltpu.make_async_copy(rows1, out_slice(0), ssem1).wait()


def kernel(inputs, table):
    idx = inputs.reshape(-1).astype(jnp.int32)
    lines = jnp.concatenate([table, table], axis=1)
    out = _embed(idx, lines)
    return out[:, : table.shape[1]].reshape(inputs.shape + (table.shape[1],))
